# Initial kernel scaffold; baseline (speedup 1.0000x reference)
import jax
import jax.numpy as jnp
from jax.experimental import pallas as pl


def kernel(eeg_emb, subject_ids, W1, b1, W2, b2):
    raise NotImplementedError("write your pallas kernel here")


# speedup vs baseline: 2.3948x; 2.3948x over previous
import jax
import jax.numpy as jnp
from jax.experimental import pallas as pl
from jax.experimental.pallas import tpu as pltpu


def _dense_body(sid_ref, x_ref, w1_ref, b1_ref, w2_ref, b2_ref, out_ref):
    s = pl.program_id(0)
    num_s = pl.num_programs(0)

    @pl.when(s == 0)
    def _():
        out_ref[...] = jnp.zeros_like(out_ref)

    h = jnp.maximum(
        jnp.dot(x_ref[...], w1_ref[0], preferred_element_type=jnp.float32)
        + b1_ref[0],
        0.0,
    )
    o = jnp.dot(h, w2_ref[0], preferred_element_type=jnp.float32) + b2_ref[0]
    mask = sid_ref[...] == s
    acc = jnp.where(mask, o, out_ref[...])

    @pl.when(s == num_s - 1)
    def _():
        norm = jnp.sqrt(jnp.sum(acc * acc, axis=1, keepdims=True))
        out_ref[...] = acc / jnp.maximum(norm, 1e-12)

    @pl.when(s != num_s - 1)
    def _():
        out_ref[...] = acc


def kernel(eeg_emb, subject_ids, W1, b1, W2, b2):
    B, eeg_dim = eeg_emb.shape
    S, _, clip_dim = W1.shape
    sid = subject_ids.astype(jnp.int32).reshape(B, 1)
    b1r = b1.reshape(S, 1, clip_dim)
    b2r = b2.reshape(S, 1, clip_dim)

    out = pl.pallas_call(
        _dense_body,
        grid=(S,),
        in_specs=[
            pl.BlockSpec((B, 1), lambda s: (0, 0)),
            pl.BlockSpec((B, eeg_dim), lambda s: (0, 0)),
            pl.BlockSpec((1, eeg_dim, clip_dim), lambda s: (s, 0, 0)),
            pl.BlockSpec((1, 1, clip_dim), lambda s: (s, 0, 0)),
            pl.BlockSpec((1, clip_dim, clip_dim), lambda s: (s, 0, 0)),
            pl.BlockSpec((1, 1, clip_dim), lambda s: (s, 0, 0)),
        ],
        out_specs=pl.BlockSpec((B, clip_dim), lambda s: (0, 0)),
        out_shape=jax.ShapeDtypeStruct((B, clip_dim), jnp.float32),
    )(sid, eeg_emb, W1, b1r, W2, b2r)
    return out
